# Initial kernel scaffold; baseline (speedup 1.0000x reference)
import jax
import jax.numpy as jnp
from jax.experimental import pallas as pl


def kernel(x, edge_index, W1, b1, gamma1, beta1, W2, b2, gamma2, beta2):
    raise NotImplementedError("write your pallas kernel here")


# speedup vs baseline: 8.0723x; 8.0723x over previous
import functools

import jax
import jax.numpy as jnp
from jax import lax
from jax.experimental import pallas as pl
from jax.experimental.pallas import tpu as pltpu
from jax.experimental.pallas import tpu_sc as plsc

NC = 2
NS = 16
NW = NC * NS
C = 128

N = 10000
D = 128
NPAD = 10240
RPT = NPAD // NS
RZB = 128


def _mesh():
    return plsc.VectorSubcoreMesh(core_axis_name="c", subcore_axis_name="s",
                                  num_cores=NC, num_subcores=NS)


def _sc_deg_body(dst3, zvec_hbm, ones_hbm, out, shared, dst_v, ones_v, zvec_v):
    c = lax.axis_index("c")
    s = lax.axis_index("s")
    wid = s * NC + c
    K = dst3.shape[1]
    pltpu.sync_copy(dst3.at[wid], dst_v)
    pltpu.sync_copy(ones_hbm, ones_v)
    pltpu.sync_copy(zvec_hbm, zvec_v)
    pltpu.sync_copy(zvec_v, shared.at[pl.ds(s * RPT, RPT)])
    plsc.subcore_barrier()

    def chunk(j, carry):
        pltpu.sync_copy(ones_v, shared.at[dst_v.at[j]], add=True)
        return carry

    lax.fori_loop(0, K, chunk, 0)
    plsc.subcore_barrier()
    pltpu.sync_copy(shared.at[pl.ds(s * RPT, RPT)], zvec_v)
    pltpu.sync_copy(zvec_v, out.at[c, pl.ds(s * RPT, RPT)])


def _sc_deg(dst3):
    K = dst3.shape[1]
    f = functools.partial(
        pl.kernel,
        out_type=jax.ShapeDtypeStruct((NC, NPAD), jnp.float32),
        mesh=_mesh(),
        scratch_types=[
            pltpu.VMEM_SHARED((NPAD,), jnp.float32),
            pltpu.VMEM((K, C), jnp.int32),
            pltpu.VMEM((C,), jnp.float32),
            pltpu.VMEM((RPT,), jnp.float32),
        ],
    )(_sc_deg_body)
    return f(dst3, jnp.zeros((RPT,), jnp.float32), jnp.ones((C,), jnp.float32))


def _sc_edge_body(hp, src3, dst3, zrows_hbm, out, shared, src_v, dst_v,
                  rows_v, gsem):
    c = lax.axis_index("c")
    s = lax.axis_index("s")
    wid = s * NC + c
    K = src3.shape[1]
    pltpu.sync_copy(src3.at[wid], src_v)
    pltpu.sync_copy(dst3.at[wid], dst_v)
    pltpu.sync_copy(zrows_hbm, rows_v)
    for z in range(RPT // RZB):
        pltpu.sync_copy(rows_v, shared.at[pl.ds(s * RPT + z * RZB, RZB)])
    plsc.subcore_barrier()

    def chunk(j, carry):
        pltpu.async_copy(hp.at[src_v.at[j]], rows_v, gsem).wait()
        pltpu.sync_copy(rows_v, shared.at[dst_v.at[j]], add=True)
        return carry

    lax.fori_loop(0, K, chunk, 0)
    plsc.subcore_barrier()
    for z in range(RPT // RZB):
        r = s * RPT + z * RZB
        pltpu.sync_copy(shared.at[pl.ds(r, RZB)], rows_v)
        pltpu.sync_copy(rows_v, out.at[c, pl.ds(r, RZB)])


def _sc_edge(hp, src3, dst3):
    K = src3.shape[1]
    f = functools.partial(
        pl.kernel,
        out_type=jax.ShapeDtypeStruct((NC, NPAD, D), jnp.float32),
        mesh=_mesh(),
        scratch_types=[
            pltpu.VMEM_SHARED((NPAD, D), jnp.float32),
            pltpu.VMEM((K, C), jnp.int32),
            pltpu.VMEM((K, C), jnp.int32),
            pltpu.VMEM((C, D), jnp.float32),
            pltpu.SemaphoreType.DMA,
        ],
    )(_sc_edge_body)
    return f(hp, src3, dst3, jnp.zeros((RZB, D), jnp.float32))


def _dinv_from(degp):
    deg = 1.0 + degp[0, :N] + degp[1, :N]
    return lax.rsqrt(deg)[:, None]


def _tc_prep_body(x_ref, w_ref, degp_ref, out_ref):
    h = jnp.dot(x_ref[...], w_ref[...], preferred_element_type=jnp.float32)
    out_ref[...] = h * _dinv_from(degp_ref[...])


def _tc_prep(x, W1, degp):
    return pl.pallas_call(
        _tc_prep_body,
        out_shape=jax.ShapeDtypeStruct((N, D), jnp.float32),
        compiler_params=pltpu.CompilerParams(vmem_limit_bytes=100 * 2**20),
    )(x, W1, degp)


def _bn_relu(z, gamma, beta):
    mean = jnp.mean(z, axis=0)
    var = jnp.mean((z - mean[None, :]) ** 2, axis=0)
    y = (z - mean[None, :]) * lax.rsqrt(var + 1e-5)[None, :]
    return jax.nn.relu(y * gamma[None, :] + beta[None, :])


def _tc_mid_body(accp_ref, hp_ref, degp_ref, g_ref, b_ref, w_ref, out_ref):
    dinv = _dinv_from(degp_ref[...])
    acc = accp_ref[0, :N, :] + accp_ref[1, :N, :]
    z = (acc + hp_ref[...]) * dinv
    y = _bn_relu(z, g_ref[...], b_ref[...])
    h2 = jnp.dot(y, w_ref[...], preferred_element_type=jnp.float32)
    out_ref[...] = h2 * dinv


def _tc_mid(accp, hp, degp, gamma, beta, W2):
    return pl.pallas_call(
        _tc_mid_body,
        out_shape=jax.ShapeDtypeStruct((N, D), jnp.float32),
        compiler_params=pltpu.CompilerParams(vmem_limit_bytes=100 * 2**20),
    )(accp, hp, degp, gamma, beta, W2)


def _tc_final_body(accp_ref, hp_ref, degp_ref, g_ref, b_ref, x_ref, out_ref):
    dinv = _dinv_from(degp_ref[...])
    acc = accp_ref[0, :N, :] + accp_ref[1, :N, :]
    z = (acc + hp_ref[...]) * dinv
    y = _bn_relu(z, g_ref[...], b_ref[...])
    out_ref[...] = y + x_ref[...]


def _tc_final(accp, hp, degp, gamma, beta, x):
    return pl.pallas_call(
        _tc_final_body,
        out_shape=jax.ShapeDtypeStruct((N, D), jnp.float32),
        compiler_params=pltpu.CompilerParams(vmem_limit_bytes=100 * 2**20),
    )(accp, hp, degp, gamma, beta, x)


def kernel(x, edge_index, W1, b1, gamma1, beta1, W2, b2, gamma2, beta2):
    E = edge_index.shape[1]
    K = -(-E // (NW * C))
    K += K % 2
    Epad = NW * K * C
    src = jnp.concatenate(
        [edge_index[0], jnp.zeros((Epad - E,), jnp.int32)]).reshape(NW, K, C)
    dst = jnp.concatenate(
        [edge_index[1], jnp.full((Epad - E,), N, jnp.int32)]).reshape(NW, K, C)

    degp = _sc_deg(dst)
    hp1 = _tc_prep(x, W1, degp)
    accp1 = _sc_edge(hp1, src, dst)
    hp2 = _tc_mid(accp1, hp1, degp, gamma1, beta1, W2)
    accp2 = _sc_edge(hp2, src, dst)
    return _tc_final(accp2, hp2, degp, gamma2, beta2, x)


# speedup vs baseline: 9.3749x; 1.1614x over previous
import functools

import jax
import jax.numpy as jnp
from jax import lax
from jax.experimental import pallas as pl
from jax.experimental.pallas import tpu as pltpu
from jax.experimental.pallas import tpu_sc as plsc

NC = 2
NS = 16
NW = NC * NS
C = 128

N = 10000
D = 128
NPAD = 10240
RPT = NPAD // NS
RZB = 128


def _mesh():
    return plsc.VectorSubcoreMesh(core_axis_name="c", subcore_axis_name="s",
                                  num_cores=NC, num_subcores=NS)


def _sc_deg_body(dst3, zvec_hbm, ones_hbm, out, shared, dst_v, ones_v, zvec_v):
    c = lax.axis_index("c")
    s = lax.axis_index("s")
    wid = s * NC + c
    K = dst3.shape[1]
    pltpu.sync_copy(dst3.at[wid], dst_v)
    pltpu.sync_copy(ones_hbm, ones_v)
    pltpu.sync_copy(zvec_hbm, zvec_v)
    pltpu.sync_copy(zvec_v, shared.at[pl.ds(s * RPT, RPT)])
    plsc.subcore_barrier()

    def chunk(j, carry):
        pltpu.sync_copy(ones_v, shared.at[dst_v.at[j]], add=True)
        return carry

    lax.fori_loop(0, K, chunk, 0)
    plsc.subcore_barrier()
    pltpu.sync_copy(shared.at[pl.ds(s * RPT, RPT)], zvec_v)
    pltpu.sync_copy(zvec_v, out.at[c, pl.ds(s * RPT, RPT)])


def _sc_deg(dst3):
    K = dst3.shape[1]
    f = functools.partial(
        pl.kernel,
        out_type=jax.ShapeDtypeStruct((NC, NPAD), jnp.float32),
        mesh=_mesh(),
        scratch_types=[
            pltpu.VMEM_SHARED((NPAD,), jnp.float32),
            pltpu.VMEM((K, C), jnp.int32),
            pltpu.VMEM((C,), jnp.float32),
            pltpu.VMEM((RPT,), jnp.float32),
        ],
    )(_sc_deg_body)
    return f(dst3, jnp.zeros((RPT,), jnp.float32), jnp.ones((C,), jnp.float32))


def _sc_edge_body(hp, idx4, zrows_hbm, out, shared, rows0, rows1, ib0, ib1,
                  gsem0, gsem1, isem0, isem1):
    c = lax.axis_index("c")
    s = lax.axis_index("s")
    wid = s * NC + c
    K = idx4.shape[1]
    rows = (rows0, rows1)
    ibuf = (ib0, ib1)
    gsem = (gsem0, gsem1)
    isem = (isem0, isem1)

    pltpu.sync_copy(zrows_hbm, rows0)
    for z in range(RPT // RZB):
        pltpu.sync_copy(rows0, shared.at[pl.ds(s * RPT + z * RZB, RZB)])
    plsc.subcore_barrier()

    pltpu.async_copy(idx4.at[wid, 0], ib0, isem0)
    pltpu.async_copy(idx4.at[wid, 1], ib1, isem1)
    pltpu.make_async_copy(idx4.at[wid, 0], ib0, isem0).wait()
    pltpu.async_copy(hp.at[ib0.at[0]], rows0, gsem0)

    def _step(j, r_cur, r_nxt, i_cur, i_nxt, g_cur, g_nxt, s_cur, s_nxt):
        @pl.when(j + 1 < K)
        def _():
            pltpu.make_async_copy(idx4.at[wid, j], i_nxt, s_nxt).wait()
            pltpu.async_copy(hp.at[i_nxt.at[0]], r_nxt, g_nxt)
        pltpu.make_async_copy(hp.at[i_cur.at[0]], r_cur, g_cur).wait()
        pltpu.sync_copy(r_cur, shared.at[i_cur.at[1]], add=True)
        @pl.when(j + 2 < K)
        def _():
            pltpu.async_copy(idx4.at[wid, j + 2], i_cur, s_cur)

    def chunk(i, carry):
        j = 2 * i
        _step(j, rows0, rows1, ib0, ib1, gsem0, gsem1, isem0, isem1)
        _step(j + 1, rows1, rows0, ib1, ib0, gsem1, gsem0, isem1, isem0)
        return carry

    lax.fori_loop(0, K // 2, chunk, 0)
    plsc.subcore_barrier()
    for z in range(RPT // RZB):
        r = s * RPT + z * RZB
        pltpu.sync_copy(shared.at[pl.ds(r, RZB)], rows0)
        pltpu.sync_copy(rows0, out.at[c, pl.ds(r, RZB)])


def _sc_edge(hp, idx4):
    K = idx4.shape[1]
    f = functools.partial(
        pl.kernel,
        out_type=jax.ShapeDtypeStruct((NC, NPAD, D), jnp.float32),
        mesh=_mesh(),
        scratch_types=[
            pltpu.VMEM_SHARED((NPAD, D), jnp.float32),
            pltpu.VMEM((C, D), jnp.float32),
            pltpu.VMEM((C, D), jnp.float32),
            pltpu.VMEM((2, C), jnp.int32),
            pltpu.VMEM((2, C), jnp.int32),
            pltpu.SemaphoreType.DMA,
            pltpu.SemaphoreType.DMA,
            pltpu.SemaphoreType.DMA,
            pltpu.SemaphoreType.DMA,
        ],
    )(_sc_edge_body)
    return f(hp, idx4, jnp.zeros((RZB, D), jnp.float32))


def _dinv_from(degp):
    deg = 1.0 + degp[0, :N] + degp[1, :N]
    return lax.rsqrt(deg)[:, None]


def _tc_prep_body(x_ref, w_ref, degp_ref, out_ref):
    h = jnp.dot(x_ref[...], w_ref[...], preferred_element_type=jnp.float32)
    out_ref[...] = h * _dinv_from(degp_ref[...])


def _tc_prep(x, W1, degp):
    return pl.pallas_call(
        _tc_prep_body,
        out_shape=jax.ShapeDtypeStruct((N, D), jnp.float32),
        compiler_params=pltpu.CompilerParams(vmem_limit_bytes=100 * 2**20),
    )(x, W1, degp)


def _bn_relu(z, gamma, beta):
    mean = jnp.mean(z, axis=0)
    var = jnp.mean((z - mean[None, :]) ** 2, axis=0)
    y = (z - mean[None, :]) * lax.rsqrt(var + 1e-5)[None, :]
    return jax.nn.relu(y * gamma[None, :] + beta[None, :])


def _tc_mid_body(accp_ref, hp_ref, degp_ref, g_ref, b_ref, w_ref, out_ref):
    dinv = _dinv_from(degp_ref[...])
    acc = accp_ref[0, :N, :] + accp_ref[1, :N, :]
    z = (acc + hp_ref[...]) * dinv
    y = _bn_relu(z, g_ref[...], b_ref[...])
    h2 = jnp.dot(y, w_ref[...], preferred_element_type=jnp.float32)
    out_ref[...] = h2 * dinv


def _tc_mid(accp, hp, degp, gamma, beta, W2):
    return pl.pallas_call(
        _tc_mid_body,
        out_shape=jax.ShapeDtypeStruct((N, D), jnp.float32),
        compiler_params=pltpu.CompilerParams(vmem_limit_bytes=100 * 2**20),
    )(accp, hp, degp, gamma, beta, W2)


def _tc_final_body(accp_ref, hp_ref, degp_ref, g_ref, b_ref, x_ref, out_ref):
    dinv = _dinv_from(degp_ref[...])
    acc = accp_ref[0, :N, :] + accp_ref[1, :N, :]
    z = (acc + hp_ref[...]) * dinv
    y = _bn_relu(z, g_ref[...], b_ref[...])
    out_ref[...] = y + x_ref[...]


def _tc_final(accp, hp, degp, gamma, beta, x):
    return pl.pallas_call(
        _tc_final_body,
        out_shape=jax.ShapeDtypeStruct((N, D), jnp.float32),
        compiler_params=pltpu.CompilerParams(vmem_limit_bytes=100 * 2**20),
    )(accp, hp, degp, gamma, beta, x)


def kernel(x, edge_index, W1, b1, gamma1, beta1, W2, b2, gamma2, beta2):
    E = edge_index.shape[1]
    K = -(-E // (NW * C))
    K += K % 2
    Epad = NW * K * C
    src = jnp.concatenate(
        [edge_index[0], jnp.zeros((Epad - E,), jnp.int32)]).reshape(NW, K, C)
    dst = jnp.concatenate(
        [edge_index[1], jnp.full((Epad - E,), N, jnp.int32)]).reshape(NW, K, C)
    idx4 = jnp.stack([src, dst], axis=2)

    degp = _sc_deg(dst)
    hp1 = _tc_prep(x, W1, degp)
    accp1 = _sc_edge(hp1, idx4)
    hp2 = _tc_mid(accp1, hp1, degp, gamma1, beta1, W2)
    accp2 = _sc_edge(hp2, idx4)
    return _tc_final(accp2, hp2, degp, gamma2, beta2, x)


# speedup vs baseline: 19.1364x; 2.0412x over previous
import functools

import jax
import jax.numpy as jnp
from jax import lax
from jax.experimental import pallas as pl
from jax.experimental.pallas import tpu as pltpu
from jax.experimental.pallas import tpu_sc as plsc

NC = 2
NS = 16
NW = NC * NS
C = 128

N = 10000
D = 128
DH = D // NC
NPAD = 10240
RPT = NPAD // NS
TPT = N // NS
RZB = 128


def _mesh():
    return plsc.VectorSubcoreMesh(core_axis_name="c", subcore_axis_name="s",
                                  num_cores=NC, num_subcores=NS)


def _sc_deg_body(dst3, zvec_hbm, ones_hbm, out, shared, dst_v, ones_v, zvec_v):
    c = lax.axis_index("c")
    s = lax.axis_index("s")
    wid = s * NC + c
    K = dst3.shape[1]
    pltpu.sync_copy(dst3.at[wid], dst_v)
    pltpu.sync_copy(ones_hbm, ones_v)
    pltpu.sync_copy(zvec_hbm, zvec_v)
    pltpu.sync_copy(zvec_v, shared.at[pl.ds(s * RPT, RPT)])
    plsc.subcore_barrier()

    def chunk(j, carry):
        pltpu.sync_copy(ones_v, shared.at[dst_v.at[j]], add=True)
        return carry

    lax.fori_loop(0, K, chunk, 0)
    plsc.subcore_barrier()
    pltpu.sync_copy(shared.at[pl.ds(s * RPT, RPT)], zvec_v)
    pltpu.sync_copy(zvec_v, out.at[c, pl.ds(s * RPT, RPT)])


def _sc_deg(dst3):
    K = dst3.shape[1]
    f = functools.partial(
        pl.kernel,
        out_type=jax.ShapeDtypeStruct((NC, NPAD), jnp.float32),
        mesh=_mesh(),
        scratch_types=[
            pltpu.VMEM_SHARED((NPAD,), jnp.float32),
            pltpu.VMEM((K, C), jnp.int32),
            pltpu.VMEM((C,), jnp.float32),
            pltpu.VMEM((RPT,), jnp.float32),
        ],
    )(_sc_deg_body)
    return f(dst3, jnp.zeros((RPT,), jnp.float32), jnp.ones((C,), jnp.float32))


def _sc_edge_body(hp2, idx3, zrows_hbm, out, tab, acc, rows0, rows1, ib0, ib1,
                  gsem0, gsem1, isem0, isem1):
    c = lax.axis_index("c")
    s = lax.axis_index("s")
    K = idx3.shape[1]

    pltpu.sync_copy(hp2.at[pl.ds(s * RPT, RPT), pl.ds(c * DH, DH)],
                    tab.at[pl.ds(s * RPT, RPT)])
    pltpu.sync_copy(zrows_hbm, rows0)
    for z in range(RPT // RZB):
        pltpu.sync_copy(rows0, acc.at[pl.ds(s * RPT + z * RZB, RZB)])
    plsc.subcore_barrier()

    pltpu.async_copy(idx3.at[s, 0], ib0, isem0)
    pltpu.async_copy(idx3.at[s, 1], ib1, isem1)
    pltpu.make_async_copy(idx3.at[s, 0], ib0, isem0).wait()
    pltpu.async_copy(tab.at[ib0.at[0]], rows0, gsem0)

    def _step(j, r_cur, r_nxt, i_cur, i_nxt, g_cur, g_nxt, s_cur, s_nxt):
        @pl.when(j + 1 < K)
        def _():
            pltpu.make_async_copy(idx3.at[s, j], i_nxt, s_nxt).wait()
            pltpu.async_copy(tab.at[i_nxt.at[0]], r_nxt, g_nxt)
        pltpu.make_async_copy(tab.at[i_cur.at[0]], r_cur, g_cur).wait()
        pltpu.sync_copy(r_cur, acc.at[i_cur.at[1]], add=True)
        @pl.when(j + 2 < K)
        def _():
            pltpu.async_copy(idx3.at[s, j + 2], i_cur, s_cur)

    def chunk(i, carry):
        j = 2 * i
        _step(j, rows0, rows1, ib0, ib1, gsem0, gsem1, isem0, isem1)
        _step(j + 1, rows1, rows0, ib1, ib0, gsem1, gsem0, isem1, isem0)
        return carry

    lax.fori_loop(0, K // 2, chunk, 0)
    plsc.subcore_barrier()
    for z in range(RPT // RZB):
        r = s * RPT + z * RZB
        pltpu.sync_copy(acc.at[pl.ds(r, RZB)], rows0)
        pltpu.sync_copy(rows0, out.at[pl.ds(r, RZB), pl.ds(c * DH, DH)])


def _sc_edge(hp2, idx3):
    K = idx3.shape[1]
    f = functools.partial(
        pl.kernel,
        out_type=jax.ShapeDtypeStruct((NPAD, D), jnp.float32),
        mesh=_mesh(),
        compiler_params=pltpu.CompilerParams(use_tc_tiling_on_sc=False),
        scratch_types=[
            pltpu.VMEM_SHARED((NPAD, DH), jnp.float32),
            pltpu.VMEM_SHARED((NPAD, DH), jnp.float32),
            pltpu.VMEM((C, DH), jnp.float32),
            pltpu.VMEM((C, DH), jnp.float32),
            pltpu.VMEM((2, C), jnp.int32),
            pltpu.VMEM((2, C), jnp.int32),
            pltpu.SemaphoreType.DMA,
            pltpu.SemaphoreType.DMA,
            pltpu.SemaphoreType.DMA,
            pltpu.SemaphoreType.DMA,
        ],
    )(_sc_edge_body)
    return f(hp2, idx3, jnp.zeros((RZB, DH), jnp.float32))


def _dinv_from(degp):
    deg = 1.0 + degp[0, :N] + degp[1, :N]
    return lax.rsqrt(deg)[:, None]


def _tc_prep_body(x_ref, w_ref, degp_ref, out_ref):
    h = jnp.dot(x_ref[...], w_ref[...], preferred_element_type=jnp.float32)
    hp = h * _dinv_from(degp_ref[...])
    out_ref[:N, :] = hp
    out_ref[N:, :] = jnp.zeros((NPAD - N, D), jnp.float32)


def _tc_prep(x, W1, degp):
    return pl.pallas_call(
        _tc_prep_body,
        out_shape=jax.ShapeDtypeStruct((NPAD, D), jnp.float32),
        compiler_params=pltpu.CompilerParams(vmem_limit_bytes=100 * 2**20),
    )(x, W1, degp)


def _bn_relu(z, gamma, beta):
    mean = jnp.mean(z, axis=0)
    var = jnp.mean((z - mean[None, :]) ** 2, axis=0)
    y = (z - mean[None, :]) * lax.rsqrt(var + 1e-5)[None, :]
    return jax.nn.relu(y * gamma[None, :] + beta[None, :])


def _acc_hp(accp_ref, hp_ref):
    return accp_ref[:N, :], hp_ref[:N, :]


def _tc_mid_body(accp_ref, hp_ref, degp_ref, g_ref, b_ref, w_ref, out_ref):
    dinv = _dinv_from(degp_ref[...])
    acc, hp = _acc_hp(accp_ref, hp_ref)
    z = (acc + hp) * dinv
    y = _bn_relu(z, g_ref[...], b_ref[...])
    h2 = jnp.dot(y, w_ref[...], preferred_element_type=jnp.float32)
    out_ref[:N, :] = h2 * dinv
    out_ref[N:, :] = jnp.zeros((NPAD - N, D), jnp.float32)


def _tc_mid(accp, hp, degp, gamma, beta, W2):
    return pl.pallas_call(
        _tc_mid_body,
        out_shape=jax.ShapeDtypeStruct((NPAD, D), jnp.float32),
        compiler_params=pltpu.CompilerParams(vmem_limit_bytes=100 * 2**20),
    )(accp, hp, degp, gamma, beta, W2)


def _tc_final_body(accp_ref, hp_ref, degp_ref, g_ref, b_ref, x_ref, out_ref):
    dinv = _dinv_from(degp_ref[...])
    acc, hp = _acc_hp(accp_ref, hp_ref)
    z = (acc + hp) * dinv
    y = _bn_relu(z, g_ref[...], b_ref[...])
    out_ref[...] = y + x_ref[...]


def _tc_final(accp, hp, degp, gamma, beta, x):
    return pl.pallas_call(
        _tc_final_body,
        out_shape=jax.ShapeDtypeStruct((N, D), jnp.float32),
        compiler_params=pltpu.CompilerParams(vmem_limit_bytes=100 * 2**20),
    )(accp, hp, degp, gamma, beta, x)


def kernel(x, edge_index, W1, b1, gamma1, beta1, W2, b2, gamma2, beta2):
    E = edge_index.shape[1]

    Kd = -(-E // (NW * C))
    dst_d = jnp.concatenate(
        [edge_index[1], jnp.full((NW * Kd * C - E,), N, jnp.int32)]
    ).reshape(NW, Kd, C)

    K = -(-E // (NS * C))
    K += K % 2
    Epad = NS * K * C
    src = jnp.concatenate(
        [edge_index[0], jnp.zeros((Epad - E,), jnp.int32)]).reshape(NS, K, C)
    dst = jnp.concatenate(
        [edge_index[1], jnp.full((Epad - E,), N, jnp.int32)]).reshape(NS, K, C)
    idx3 = jnp.stack([src, dst], axis=2)

    degp = _sc_deg(dst_d)
    hp1 = _tc_prep(x, W1, degp)
    accp1 = _sc_edge(hp1, idx3)
    hp2 = _tc_mid(accp1, hp1, degp, gamma1, beta1, W2)
    accp2 = _sc_edge(hp2, idx3)
    return _tc_final(accp2, hp2, degp, gamma2, beta2, x)


# speedup vs baseline: 28.3322x; 1.4805x over previous
import functools

import jax
import jax.numpy as jnp
from jax import lax
from jax.experimental import pallas as pl
from jax.experimental.pallas import tpu as pltpu
from jax.experimental.pallas import tpu_sc as plsc

NC = 2
NS = 16
NW = NC * NS
C = 128

N = 10000
D = 128
DH = D // NC
NPAD = 10240
RPT = NPAD // NS
TPT = N // NS
RZB = 128


def _mesh():
    return plsc.VectorSubcoreMesh(core_axis_name="c", subcore_axis_name="s",
                                  num_cores=NC, num_subcores=NS)


def _sc_deg_body(dst3, zvec_hbm, ones_hbm, out, shared, dst_v, ones_v, zvec_v):
    c = lax.axis_index("c")
    s = lax.axis_index("s")
    wid = s * NC + c
    K = dst3.shape[1]
    pltpu.sync_copy(dst3.at[wid], dst_v)
    pltpu.sync_copy(ones_hbm, ones_v)
    pltpu.sync_copy(zvec_hbm, zvec_v)
    pltpu.sync_copy(zvec_v, shared.at[pl.ds(s * RPT, RPT)])
    plsc.subcore_barrier()

    def chunk(j, carry):
        pltpu.sync_copy(ones_v, shared.at[dst_v.at[j]], add=True)
        return carry

    lax.fori_loop(0, K, chunk, 0)
    plsc.subcore_barrier()
    pltpu.sync_copy(shared.at[pl.ds(s * RPT, RPT)], zvec_v)
    pltpu.sync_copy(zvec_v, out.at[c, pl.ds(s * RPT, RPT)])


def _sc_deg(dst3):
    K = dst3.shape[1]
    f = functools.partial(
        pl.kernel,
        out_type=jax.ShapeDtypeStruct((NC, NPAD), jnp.float32),
        mesh=_mesh(),
        scratch_types=[
            pltpu.VMEM_SHARED((NPAD,), jnp.float32),
            pltpu.VMEM((K, C), jnp.int32),
            pltpu.VMEM((C,), jnp.float32),
            pltpu.VMEM((RPT,), jnp.float32),
        ],
    )(_sc_deg_body)
    return f(dst3, jnp.zeros((RPT,), jnp.float32), jnp.ones((C,), jnp.float32))


def _sc_edge_body(hp2, idxP, zrows_hbm, out, tab, acc,
                  r0a, r1a, r0b, r1b, isrcA, idstA, isrcB, idstB,
                  gsemA, gsemB, ssemA, ssemB, isemSA, isemSB, isemDA, isemDB):
    c = lax.axis_index("c")
    s = lax.axis_index("s")
    K = idxP.shape[2]
    P = K // 2

    pltpu.sync_copy(hp2.at[pl.ds(s * RPT, RPT), pl.ds(c * DH, DH)],
                    tab.at[pl.ds(s * RPT, RPT)])
    pltpu.sync_copy(zrows_hbm, r0a)
    for z in range(RPT // RZB):
        pltpu.sync_copy(r0a, acc.at[pl.ds(s * RPT + z * RZB, RZB)])
    plsc.subcore_barrier()

    def fire_src(p, isrc, isem):
        pltpu.async_copy(idxP.at[0, s, pl.ds(2 * p, 2)], isrc, isem)

    def fire_dst(p, idst, isem):
        pltpu.async_copy(idxP.at[1, s, pl.ds(2 * p, 2)], idst, isem)

    def w(sem, srcref, dstref):
        pltpu.make_async_copy(srcref, dstref, sem).wait()

    def pair_body(p, r0x, r1x, r0y, r1y, isrcX, idstX, isrcY, idstY,
                  gsemX, gsemY, ssemX, ssemY, isemSX, isemSY,
                  isemDX, isemDY, first):
        j = 2 * p
        w(gsemX, tab.at[isrcX.at[0]], r0x)
        w(gsemX, tab.at[isrcX.at[1]], r1x)
        @pl.when(j + 4 < K)
        def _():
            fire_src(p + 2, isrcX, isemSX)
        w(isemDX, idxP.at[1, s, pl.ds(j, 2)], idstX)
        pltpu.async_copy(r0x, acc.at[idstX.at[0]], ssemX, add=True)
        pltpu.async_copy(r1x, acc.at[idstX.at[1]], ssemX, add=True)
        if not first:
            w(ssemY, r0y, acc.at[idstY.at[0]])
            w(ssemY, r1y, acc.at[idstY.at[1]])
        @pl.when(j + 2 < K)
        def _():
            w(isemSY, idxP.at[0, s, pl.ds(j + 2, 2)], isrcY)
            pltpu.async_copy(tab.at[isrcY.at[0]], r0y, gsemY)
            pltpu.async_copy(tab.at[isrcY.at[1]], r1y, gsemY)
        if not first:
            @pl.when(j + 2 < K)
            def _():
                fire_dst(p + 1, idstY, isemDY)

    A = (r0a, r1a, r0b, r1b, isrcA, idstA, isrcB, idstB,
         gsemA, gsemB, ssemA, ssemB, isemSA, isemSB, isemDA, isemDB)
    B = (r0b, r1b, r0a, r1a, isrcB, idstB, isrcA, idstA,
         gsemB, gsemA, ssemB, ssemA, isemSB, isemSA, isemDB, isemDA)

    fire_src(0, isrcA, isemSA)
    fire_dst(0, idstA, isemDA)
    fire_src(1, isrcB, isemSB)
    fire_dst(1, idstB, isemDB)
    w(isemSA, idxP.at[0, s, pl.ds(0, 2)], isrcA)
    pltpu.async_copy(tab.at[isrcA.at[0]], r0a, gsemA)
    pltpu.async_copy(tab.at[isrcA.at[1]], r1a, gsemA)

    pair_body(0, *A, True)

    def loop(i, carry):
        pair_body(2 * i + 1, *B, False)
        pair_body(2 * i + 2, *A, False)
        return carry

    lax.fori_loop(0, (P - 1) // 2, loop, 0)
    w(ssemA, r0a, acc.at[idstA.at[0]])
    w(ssemA, r1a, acc.at[idstA.at[1]])

    plsc.subcore_barrier()
    for z in range(RPT // RZB):
        r = s * RPT + z * RZB
        pltpu.sync_copy(acc.at[pl.ds(r, RZB)], r0a)
        pltpu.sync_copy(r0a, out.at[pl.ds(r, RZB), pl.ds(c * DH, DH)])


def _sc_edge(hp2, idxP):
    f = functools.partial(
        pl.kernel,
        out_type=jax.ShapeDtypeStruct((NPAD, D), jnp.float32),
        mesh=_mesh(),
        compiler_params=pltpu.CompilerParams(use_tc_tiling_on_sc=False),
        scratch_types=[
            pltpu.VMEM_SHARED((NPAD, DH), jnp.float32),
            pltpu.VMEM_SHARED((NPAD, DH), jnp.float32),
            pltpu.VMEM((C, DH), jnp.float32),
            pltpu.VMEM((C, DH), jnp.float32),
            pltpu.VMEM((C, DH), jnp.float32),
            pltpu.VMEM((C, DH), jnp.float32),
            pltpu.VMEM((2, C), jnp.int32),
            pltpu.VMEM((2, C), jnp.int32),
            pltpu.VMEM((2, C), jnp.int32),
            pltpu.VMEM((2, C), jnp.int32),
        ] + [pltpu.SemaphoreType.DMA] * 8,
    )(_sc_edge_body)
    return f(hp2, idxP, jnp.zeros((RZB, DH), jnp.float32))


def _dinv_from(degp):
    deg = 1.0 + degp[0, :N] + degp[1, :N]
    return lax.rsqrt(deg)[:, None]


def _tc_prep_body(x_ref, w_ref, degp_ref, out_ref):
    h = jnp.dot(x_ref[...], w_ref[...], preferred_element_type=jnp.float32)
    hp = h * _dinv_from(degp_ref[...])
    out_ref[:N, :] = hp
    out_ref[N:, :] = jnp.zeros((NPAD - N, D), jnp.float32)


def _tc_prep(x, W1, degp):
    return pl.pallas_call(
        _tc_prep_body,
        out_shape=jax.ShapeDtypeStruct((NPAD, D), jnp.float32),
        compiler_params=pltpu.CompilerParams(vmem_limit_bytes=100 * 2**20),
    )(x, W1, degp)


def _bn_relu(z, gamma, beta):
    mean = jnp.mean(z, axis=0)
    var = jnp.mean((z - mean[None, :]) ** 2, axis=0)
    y = (z - mean[None, :]) * lax.rsqrt(var + 1e-5)[None, :]
    return jax.nn.relu(y * gamma[None, :] + beta[None, :])


def _acc_hp(accp_ref, hp_ref):
    return accp_ref[:N, :], hp_ref[:N, :]


def _tc_mid_body(accp_ref, hp_ref, degp_ref, g_ref, b_ref, w_ref, out_ref):
    dinv = _dinv_from(degp_ref[...])
    acc, hp = _acc_hp(accp_ref, hp_ref)
    z = (acc + hp) * dinv
    y = _bn_relu(z, g_ref[...], b_ref[...])
    h2 = jnp.dot(y, w_ref[...], preferred_element_type=jnp.float32)
    out_ref[:N, :] = h2 * dinv
    out_ref[N:, :] = jnp.zeros((NPAD - N, D), jnp.float32)


def _tc_mid(accp, hp, degp, gamma, beta, W2):
    return pl.pallas_call(
        _tc_mid_body,
        out_shape=jax.ShapeDtypeStruct((NPAD, D), jnp.float32),
        compiler_params=pltpu.CompilerParams(vmem_limit_bytes=100 * 2**20),
    )(accp, hp, degp, gamma, beta, W2)


def _tc_final_body(accp_ref, hp_ref, degp_ref, g_ref, b_ref, x_ref, out_ref):
    dinv = _dinv_from(degp_ref[...])
    acc, hp = _acc_hp(accp_ref, hp_ref)
    z = (acc + hp) * dinv
    y = _bn_relu(z, g_ref[...], b_ref[...])
    out_ref[...] = y + x_ref[...]


def _tc_final(accp, hp, degp, gamma, beta, x):
    return pl.pallas_call(
        _tc_final_body,
        out_shape=jax.ShapeDtypeStruct((N, D), jnp.float32),
        compiler_params=pltpu.CompilerParams(vmem_limit_bytes=100 * 2**20),
    )(accp, hp, degp, gamma, beta, x)


def kernel(x, edge_index, W1, b1, gamma1, beta1, W2, b2, gamma2, beta2):
    E = edge_index.shape[1]

    K = -(-E // (NS * C))
    K += (2 - K) % 4
    pad = NS * K * C - E
    padcol = jnp.stack([jnp.zeros((pad,), jnp.int32),
                        jnp.full((pad,), N, jnp.int32)])
    idxP = jnp.concatenate([edge_index, padcol], axis=1).reshape(2, NS, K, C)
    dst_d = idxP[1].reshape(NW, K // 2, C)

    degp = _sc_deg(dst_d)
    hp1 = _tc_prep(x, W1, degp)
    accp1 = _sc_edge(hp1, idxP)
    hp2 = _tc_mid(accp1, hp1, degp, gamma1, beta1, W2)
    accp2 = _sc_edge(hp2, idxP)
    return _tc_final(accp2, hp2, degp, gamma2, beta2, x)


# speedup vs baseline: 28.7193x; 1.0137x over previous
import functools

import jax
import jax.numpy as jnp
from jax import lax
from jax.experimental import pallas as pl
from jax.experimental.pallas import tpu as pltpu
from jax.experimental.pallas import tpu_sc as plsc

NC = 2
NS = 16
NW = NC * NS
C = 128

N = 10000
D = 128
DH = D // NC
NPAD = 10240
RPT = NPAD // NS
TPT = N // NS
RZB = 128


def _mesh():
    return plsc.VectorSubcoreMesh(core_axis_name="c", subcore_axis_name="s",
                                  num_cores=NC, num_subcores=NS)


DEG_SLAB = 80


def _sc_deg_body(ei2, zvec_hbm, ones_hbm, out, shared, dst_v, ones_v, zvec_v):
    c = lax.axis_index("c")
    s = lax.axis_index("s")
    wid = s * NC + c
    rows = ei2.shape[1]
    ntail = rows - (NW - 1) * DEG_SLAB
    pltpu.sync_copy(ones_hbm, ones_v)
    pltpu.sync_copy(zvec_hbm, zvec_v)
    pltpu.sync_copy(zvec_v, shared.at[pl.ds(s * RPT, RPT)])

    @pl.when(wid < NW - 1)
    def _():
        pltpu.sync_copy(ei2.at[1, pl.ds(wid * DEG_SLAB, DEG_SLAB)], dst_v)

    @pl.when(wid == NW - 1)
    def _():
        pltpu.sync_copy(ei2.at[1, pl.ds((NW - 1) * DEG_SLAB, ntail)],
                        dst_v.at[pl.ds(0, ntail)])
    plsc.subcore_barrier()

    nj = jnp.where(wid == NW - 1, ntail, DEG_SLAB)

    def chunk(j, carry):
        pltpu.sync_copy(ones_v, shared.at[dst_v.at[j]], add=True)
        return carry

    lax.fori_loop(0, nj, chunk, 0)
    plsc.subcore_barrier()
    pltpu.sync_copy(shared.at[pl.ds(s * RPT, RPT)], zvec_v)
    pltpu.sync_copy(zvec_v, out.at[c, pl.ds(s * RPT, RPT)])


def _sc_deg(ei2):
    f = functools.partial(
        pl.kernel,
        out_type=jax.ShapeDtypeStruct((NC, NPAD), jnp.float32),
        mesh=_mesh(),
        scratch_types=[
            pltpu.VMEM_SHARED((NPAD,), jnp.float32),
            pltpu.VMEM((DEG_SLAB, C), jnp.int32),
            pltpu.VMEM((C,), jnp.float32),
            pltpu.VMEM((RPT,), jnp.float32),
        ],
    )(_sc_deg_body)
    return f(ei2, jnp.zeros((RPT,), jnp.float32), jnp.ones((C,), jnp.float32))


def _sc_edge_body(hp2, idxP, zrows_hbm, out, tab, acc,
                  r0a, r1a, r0b, r1b, isrcA, idstA, isrcB, idstB,
                  gsemA, gsemB, ssemA, ssemB, isemSA, isemSB, isemDA, isemDB):
    c = lax.axis_index("c")
    s = lax.axis_index("s")
    K = idxP.shape[2]
    P = K // 2

    pltpu.sync_copy(hp2.at[pl.ds(s * RPT, RPT), pl.ds(c * DH, DH)],
                    tab.at[pl.ds(s * RPT, RPT)])
    pltpu.sync_copy(zrows_hbm, acc.at[pl.ds(s * RPT, RPT)])
    plsc.subcore_barrier()

    def fire_src(p, isrc, isem):
        pltpu.async_copy(idxP.at[0, s, pl.ds(2 * p, 2)], isrc, isem)

    def fire_dst(p, idst, isem):
        pltpu.async_copy(idxP.at[1, s, pl.ds(2 * p, 2)], idst, isem)

    def w(sem, srcref, dstref):
        pltpu.make_async_copy(srcref, dstref, sem).wait()

    def pair_body(p, r0x, r1x, r0y, r1y, isrcX, idstX, isrcY, idstY,
                  gsemX, gsemY, ssemX, ssemY, isemSX, isemSY,
                  isemDX, isemDY, first):
        j = 2 * p
        w(gsemX, tab.at[isrcX.at[0]], r0x)
        w(gsemX, tab.at[isrcX.at[1]], r1x)
        @pl.when(j + 4 < K)
        def _():
            fire_src(p + 2, isrcX, isemSX)
        w(isemDX, idxP.at[1, s, pl.ds(j, 2)], idstX)
        pltpu.async_copy(r0x, acc.at[idstX.at[0]], ssemX, add=True)
        pltpu.async_copy(r1x, acc.at[idstX.at[1]], ssemX, add=True)
        if not first:
            w(ssemY, r0y, acc.at[idstY.at[0]])
            w(ssemY, r1y, acc.at[idstY.at[1]])
        @pl.when(j + 2 < K)
        def _():
            w(isemSY, idxP.at[0, s, pl.ds(j + 2, 2)], isrcY)
            pltpu.async_copy(tab.at[isrcY.at[0]], r0y, gsemY)
            pltpu.async_copy(tab.at[isrcY.at[1]], r1y, gsemY)
        if not first:
            @pl.when(j + 2 < K)
            def _():
                fire_dst(p + 1, idstY, isemDY)

    A = (r0a, r1a, r0b, r1b, isrcA, idstA, isrcB, idstB,
         gsemA, gsemB, ssemA, ssemB, isemSA, isemSB, isemDA, isemDB)
    B = (r0b, r1b, r0a, r1a, isrcB, idstB, isrcA, idstA,
         gsemB, gsemA, ssemB, ssemA, isemSB, isemSA, isemDB, isemDA)

    fire_src(0, isrcA, isemSA)
    fire_dst(0, idstA, isemDA)
    fire_src(1, isrcB, isemSB)
    fire_dst(1, idstB, isemDB)
    w(isemSA, idxP.at[0, s, pl.ds(0, 2)], isrcA)
    pltpu.async_copy(tab.at[isrcA.at[0]], r0a, gsemA)
    pltpu.async_copy(tab.at[isrcA.at[1]], r1a, gsemA)

    pair_body(0, *A, True)

    def loop(i, carry):
        pair_body(2 * i + 1, *B, False)
        pair_body(2 * i + 2, *A, False)
        return carry

    lax.fori_loop(0, (P - 1) // 2, loop, 0)
    w(ssemA, r0a, acc.at[idstA.at[0]])
    w(ssemA, r1a, acc.at[idstA.at[1]])

    plsc.subcore_barrier()
    rs = (r0a, r1a, r0b, r1b)
    nz = RPT // RZB
    pltpu.async_copy(acc.at[pl.ds(s * RPT, RZB)], rs[0], gsemA)
    for z in range(nz):
        r = s * RPT + z * RZB
        w(gsemA, acc.at[pl.ds(r, RZB)], rs[z % 4])
        if z + 1 < nz:
            if z + 1 >= 4:
                w(ssemA, rs[(z + 1) % 4],
                  out.at[pl.ds(s * RPT, RZB), pl.ds(c * DH, DH)])
            pltpu.async_copy(acc.at[pl.ds(r + RZB, RZB)], rs[(z + 1) % 4],
                             gsemA)
        pltpu.async_copy(rs[z % 4], out.at[pl.ds(r, RZB), pl.ds(c * DH, DH)],
                         ssemA)
    for z in range(min(nz, 4) if nz < 4 else 4):
        w(ssemA, rs[0], out.at[pl.ds(s * RPT, RZB), pl.ds(c * DH, DH)])


def _sc_edge(hp2, idxP):
    f = functools.partial(
        pl.kernel,
        out_type=jax.ShapeDtypeStruct((NPAD, D), jnp.float32),
        mesh=_mesh(),
        compiler_params=pltpu.CompilerParams(use_tc_tiling_on_sc=False),
        scratch_types=[
            pltpu.VMEM_SHARED((NPAD, DH), jnp.float32),
            pltpu.VMEM_SHARED((NPAD, DH), jnp.float32),
            pltpu.VMEM((C, DH), jnp.float32),
            pltpu.VMEM((C, DH), jnp.float32),
            pltpu.VMEM((C, DH), jnp.float32),
            pltpu.VMEM((C, DH), jnp.float32),
            pltpu.VMEM((2, C), jnp.int32),
            pltpu.VMEM((2, C), jnp.int32),
            pltpu.VMEM((2, C), jnp.int32),
            pltpu.VMEM((2, C), jnp.int32),
        ] + [pltpu.SemaphoreType.DMA] * 8,
    )(_sc_edge_body)
    return f(hp2, idxP, jnp.zeros((RPT, DH), jnp.float32))


def _dinv_from(degp):
    deg = 1.0 + degp[0, :N] + degp[1, :N]
    return lax.rsqrt(deg)[:, None]


def _tc_prep_body(x_ref, w_ref, degp_ref, out_ref):
    h = jnp.dot(x_ref[...], w_ref[...], preferred_element_type=jnp.float32)
    hp = h * _dinv_from(degp_ref[...])
    out_ref[:N, :] = hp
    out_ref[N:, :] = jnp.zeros((NPAD - N, D), jnp.float32)


def _tc_prep(x, W1, degp):
    return pl.pallas_call(
        _tc_prep_body,
        out_shape=jax.ShapeDtypeStruct((NPAD, D), jnp.float32),
        compiler_params=pltpu.CompilerParams(vmem_limit_bytes=100 * 2**20),
    )(x, W1, degp)


def _bn_relu(z, gamma, beta):
    mean = jnp.mean(z, axis=0)
    var = jnp.mean((z - mean[None, :]) ** 2, axis=0)
    y = (z - mean[None, :]) * lax.rsqrt(var + 1e-5)[None, :]
    return jax.nn.relu(y * gamma[None, :] + beta[None, :])


def _acc_hp(accp_ref, hp_ref):
    return accp_ref[:N, :], hp_ref[:N, :]


def _tc_mid_body(accp_ref, hp_ref, degp_ref, g_ref, b_ref, w_ref, out_ref):
    dinv = _dinv_from(degp_ref[...])
    acc, hp = _acc_hp(accp_ref, hp_ref)
    z = (acc + hp) * dinv
    y = _bn_relu(z, g_ref[...], b_ref[...])
    h2 = jnp.dot(y, w_ref[...], preferred_element_type=jnp.float32)
    out_ref[:N, :] = h2 * dinv
    out_ref[N:, :] = jnp.zeros((NPAD - N, D), jnp.float32)


def _tc_mid(accp, hp, degp, gamma, beta, W2):
    return pl.pallas_call(
        _tc_mid_body,
        out_shape=jax.ShapeDtypeStruct((NPAD, D), jnp.float32),
        compiler_params=pltpu.CompilerParams(vmem_limit_bytes=100 * 2**20),
    )(accp, hp, degp, gamma, beta, W2)


def _tc_final_body(accp_ref, hp_ref, degp_ref, g_ref, b_ref, x_ref, out_ref):
    dinv = _dinv_from(degp_ref[...])
    acc, hp = _acc_hp(accp_ref, hp_ref)
    z = (acc + hp) * dinv
    y = _bn_relu(z, g_ref[...], b_ref[...])
    out_ref[...] = y + x_ref[...]


def _tc_final(accp, hp, degp, gamma, beta, x):
    return pl.pallas_call(
        _tc_final_body,
        out_shape=jax.ShapeDtypeStruct((N, D), jnp.float32),
        compiler_params=pltpu.CompilerParams(vmem_limit_bytes=100 * 2**20),
    )(accp, hp, degp, gamma, beta, x)


def kernel(x, edge_index, W1, b1, gamma1, beta1, W2, b2, gamma2, beta2):
    E = edge_index.shape[1]

    K = -(-E // (NS * C))
    K += (2 - K) % 4
    pad = NS * K * C - E
    padcol = jnp.stack([jnp.zeros((pad,), jnp.int32),
                        jnp.full((pad,), N, jnp.int32)])
    idxP = jnp.concatenate([edge_index, padcol], axis=1).reshape(2, NS, K, C)

    degp = _sc_deg(edge_index.reshape(2, E // C, C))
    hp1 = _tc_prep(x, W1, degp)
    accp1 = _sc_edge(hp1, idxP)
    hp2 = _tc_mid(accp1, hp1, degp, gamma1, beta1, W2)
    accp2 = _sc_edge(hp2, idxP)
    return _tc_final(accp2, hp2, degp, gamma2, beta2, x)


# speedup vs baseline: 29.1003x; 1.0133x over previous
import functools

import jax
import jax.numpy as jnp
from jax import lax
from jax.experimental import pallas as pl
from jax.experimental.pallas import tpu as pltpu
from jax.experimental.pallas import tpu_sc as plsc

NC = 2
NS = 16
NW = NC * NS
C = 128

N = 10000
D = 128
DH = D // NC
NPAD = 10240
RPT = NPAD // NS
TPT = N // NS
RZB = 128


def _mesh():
    return plsc.VectorSubcoreMesh(core_axis_name="c", subcore_axis_name="s",
                                  num_cores=NC, num_subcores=NS)


DEG_SLAB = 80


def _sc_deg_body(ei2, zvec_hbm, ones_hbm, out, shared, dst_v, ones_v, zvec_v):
    c = lax.axis_index("c")
    s = lax.axis_index("s")
    wid = s * NC + c
    rows = ei2.shape[1]
    ntail = rows - (NW - 1) * DEG_SLAB
    pltpu.sync_copy(ones_hbm, ones_v)
    pltpu.sync_copy(zvec_hbm, zvec_v)
    pltpu.sync_copy(zvec_v, shared.at[pl.ds(s * RPT, RPT)])

    @pl.when(wid < NW - 1)
    def _():
        pltpu.sync_copy(ei2.at[1, pl.ds(wid * DEG_SLAB, DEG_SLAB)], dst_v)

    @pl.when(wid == NW - 1)
    def _():
        pltpu.sync_copy(ei2.at[1, pl.ds((NW - 1) * DEG_SLAB, ntail)],
                        dst_v.at[pl.ds(0, ntail)])
    plsc.subcore_barrier()

    nj = jnp.where(wid == NW - 1, ntail, DEG_SLAB)

    def chunk(j, carry):
        pltpu.sync_copy(ones_v, shared.at[dst_v.at[j]], add=True)
        return carry

    lax.fori_loop(0, nj, chunk, 0)
    plsc.subcore_barrier()
    pltpu.sync_copy(shared.at[pl.ds(s * RPT, RPT)], zvec_v)
    pltpu.sync_copy(zvec_v, out.at[c, pl.ds(s * RPT, RPT)])


def _sc_deg(ei2):
    f = functools.partial(
        pl.kernel,
        out_type=jax.ShapeDtypeStruct((NC, NPAD), jnp.float32),
        mesh=_mesh(),
        scratch_types=[
            pltpu.VMEM_SHARED((NPAD,), jnp.float32),
            pltpu.VMEM((DEG_SLAB, C), jnp.int32),
            pltpu.VMEM((C,), jnp.float32),
            pltpu.VMEM((RPT,), jnp.float32),
        ],
    )(_sc_deg_body)
    return f(ei2, jnp.zeros((RPT,), jnp.float32), jnp.ones((C,), jnp.float32))


def _sc_edge_body(hp2, idxP, out, tab, acc,
                  r0a, r1a, r0b, r1b, isrcA, idstA, isrcB, idstB,
                  gsemA, gsemB, ssemA, ssemB, isemSA, isemSB, isemDA, isemDB):
    c = lax.axis_index("c")
    s = lax.axis_index("s")
    K = idxP.shape[2]
    P = K // 2

    pltpu.sync_copy(hp2.at[pl.ds(s * RPT, RPT), pl.ds(c * DH, DH)],
                    tab.at[pl.ds(s * RPT, RPT)])
    pltpu.sync_copy(hp2.at[pl.ds(s * RPT, RPT), pl.ds(c * DH, DH)],
                    acc.at[pl.ds(s * RPT, RPT)])
    plsc.subcore_barrier()

    def fire_src(p, isrc, isem):
        pltpu.async_copy(idxP.at[0, s, pl.ds(2 * p, 2)], isrc, isem)

    def fire_dst(p, idst, isem):
        pltpu.async_copy(idxP.at[1, s, pl.ds(2 * p, 2)], idst, isem)

    def w(sem, srcref, dstref):
        pltpu.make_async_copy(srcref, dstref, sem).wait()

    def pair_body(p, r0x, r1x, r0y, r1y, isrcX, idstX, isrcY, idstY,
                  gsemX, gsemY, ssemX, ssemY, isemSX, isemSY,
                  isemDX, isemDY, first):
        j = 2 * p
        w(gsemX, tab.at[isrcX.at[0]], r0x)
        w(gsemX, tab.at[isrcX.at[1]], r1x)
        @pl.when(j + 4 < K)
        def _():
            fire_src(p + 2, isrcX, isemSX)
        w(isemDX, idxP.at[1, s, pl.ds(j, 2)], idstX)
        pltpu.async_copy(r0x, acc.at[idstX.at[0]], ssemX, add=True)
        pltpu.async_copy(r1x, acc.at[idstX.at[1]], ssemX, add=True)
        if not first:
            w(ssemY, r0y, acc.at[idstY.at[0]])
            w(ssemY, r1y, acc.at[idstY.at[1]])
        @pl.when(j + 2 < K)
        def _():
            w(isemSY, idxP.at[0, s, pl.ds(j + 2, 2)], isrcY)
            pltpu.async_copy(tab.at[isrcY.at[0]], r0y, gsemY)
            pltpu.async_copy(tab.at[isrcY.at[1]], r1y, gsemY)
        if not first:
            @pl.when(j + 2 < K)
            def _():
                fire_dst(p + 1, idstY, isemDY)

    A = (r0a, r1a, r0b, r1b, isrcA, idstA, isrcB, idstB,
         gsemA, gsemB, ssemA, ssemB, isemSA, isemSB, isemDA, isemDB)
    B = (r0b, r1b, r0a, r1a, isrcB, idstB, isrcA, idstA,
         gsemB, gsemA, ssemB, ssemA, isemSB, isemSA, isemDB, isemDA)

    fire_src(0, isrcA, isemSA)
    fire_dst(0, idstA, isemDA)
    fire_src(1, isrcB, isemSB)
    fire_dst(1, idstB, isemDB)
    w(isemSA, idxP.at[0, s, pl.ds(0, 2)], isrcA)
    pltpu.async_copy(tab.at[isrcA.at[0]], r0a, gsemA)
    pltpu.async_copy(tab.at[isrcA.at[1]], r1a, gsemA)

    pair_body(0, *A, True)

    def loop(i, carry):
        pair_body(2 * i + 1, *B, False)
        pair_body(2 * i + 2, *A, False)
        return carry

    lax.fori_loop(0, (P - 1) // 2, loop, 0)
    w(ssemA, r0a, acc.at[idstA.at[0]])
    w(ssemA, r1a, acc.at[idstA.at[1]])

    plsc.subcore_barrier()
    rs = (r0a, r1a, r0b, r1b)
    nz = RPT // RZB
    pltpu.async_copy(acc.at[pl.ds(s * RPT, RZB)], rs[0], gsemA)
    for z in range(nz):
        r = s * RPT + z * RZB
        w(gsemA, acc.at[pl.ds(r, RZB)], rs[z % 4])
        if z + 1 < nz:
            if z + 1 >= 4:
                w(ssemA, rs[(z + 1) % 4],
                  out.at[pl.ds(s * RPT, RZB), pl.ds(c * DH, DH)])
            pltpu.async_copy(acc.at[pl.ds(r + RZB, RZB)], rs[(z + 1) % 4],
                             gsemA)
        pltpu.async_copy(rs[z % 4], out.at[pl.ds(r, RZB), pl.ds(c * DH, DH)],
                         ssemA)
    for z in range(min(nz, 4) if nz < 4 else 4):
        w(ssemA, rs[0], out.at[pl.ds(s * RPT, RZB), pl.ds(c * DH, DH)])


def _sc_edge(hp2, idxP):
    f = functools.partial(
        pl.kernel,
        out_type=jax.ShapeDtypeStruct((NPAD, D), jnp.float32),
        mesh=_mesh(),
        compiler_params=pltpu.CompilerParams(use_tc_tiling_on_sc=False),
        scratch_types=[
            pltpu.VMEM_SHARED((NPAD, DH), jnp.float32),
            pltpu.VMEM_SHARED((NPAD, DH), jnp.float32),
            pltpu.VMEM((C, DH), jnp.float32),
            pltpu.VMEM((C, DH), jnp.float32),
            pltpu.VMEM((C, DH), jnp.float32),
            pltpu.VMEM((C, DH), jnp.float32),
            pltpu.VMEM((2, C), jnp.int32),
            pltpu.VMEM((2, C), jnp.int32),
            pltpu.VMEM((2, C), jnp.int32),
            pltpu.VMEM((2, C), jnp.int32),
        ] + [pltpu.SemaphoreType.DMA] * 8,
    )(_sc_edge_body)
    return f(hp2, idxP)


def _dinv_from(degp):
    deg = 1.0 + degp[0, :N] + degp[1, :N]
    return lax.rsqrt(deg)[:, None]


def _tc_prep_body(x_ref, w_ref, degp_ref, out_ref):
    h = jnp.dot(x_ref[...], w_ref[...], preferred_element_type=jnp.float32)
    hp = h * _dinv_from(degp_ref[...])
    out_ref[:N, :] = hp
    out_ref[N:, :] = jnp.zeros((NPAD - N, D), jnp.float32)


def _tc_prep(x, W1, degp):
    return pl.pallas_call(
        _tc_prep_body,
        out_shape=jax.ShapeDtypeStruct((NPAD, D), jnp.float32),
        compiler_params=pltpu.CompilerParams(vmem_limit_bytes=100 * 2**20),
    )(x, W1, degp)


def _bn_relu(z, gamma, beta):
    mean = jnp.mean(z, axis=0)
    var = jnp.mean((z - mean[None, :]) ** 2, axis=0)
    y = (z - mean[None, :]) * lax.rsqrt(var + 1e-5)[None, :]
    return jax.nn.relu(y * gamma[None, :] + beta[None, :])


def _tc_mid_body(accp_ref, degp_ref, g_ref, b_ref, w_ref, out_ref):
    dinv = _dinv_from(degp_ref[...])
    z = accp_ref[:N, :] * dinv
    y = _bn_relu(z, g_ref[...], b_ref[...])
    h2 = jnp.dot(y, w_ref[...], preferred_element_type=jnp.float32)
    out_ref[:N, :] = h2 * dinv
    out_ref[N:, :] = jnp.zeros((NPAD - N, D), jnp.float32)


def _tc_mid(accp, degp, gamma, beta, W2):
    return pl.pallas_call(
        _tc_mid_body,
        out_shape=jax.ShapeDtypeStruct((NPAD, D), jnp.float32),
        compiler_params=pltpu.CompilerParams(vmem_limit_bytes=100 * 2**20),
    )(accp, degp, gamma, beta, W2)


def _tc_final_body(accp_ref, degp_ref, g_ref, b_ref, x_ref, out_ref):
    dinv = _dinv_from(degp_ref[...])
    z = accp_ref[:N, :] * dinv
    y = _bn_relu(z, g_ref[...], b_ref[...])
    out_ref[...] = y + x_ref[...]


def _tc_final(accp, degp, gamma, beta, x):
    return pl.pallas_call(
        _tc_final_body,
        out_shape=jax.ShapeDtypeStruct((N, D), jnp.float32),
        compiler_params=pltpu.CompilerParams(vmem_limit_bytes=100 * 2**20),
    )(accp, degp, gamma, beta, x)


def kernel(x, edge_index, W1, b1, gamma1, beta1, W2, b2, gamma2, beta2):
    E = edge_index.shape[1]

    K = -(-E // (NS * C))
    K += (2 - K) % 4
    pad = NS * K * C - E
    padcol = jnp.stack([jnp.zeros((pad,), jnp.int32),
                        jnp.full((pad,), N, jnp.int32)])
    idxP = jnp.concatenate([edge_index, padcol], axis=1).reshape(2, NS, K, C)

    degp = _sc_deg(edge_index.reshape(2, E // C, C))
    hp1 = _tc_prep(x, W1, degp)
    accp1 = _sc_edge(hp1, idxP)
    hp2 = _tc_mid(accp1, degp, gamma1, beta1, W2)
    accp2 = _sc_edge(hp2, idxP)
    return _tc_final(accp2, degp, gamma2, beta2, x)


# speedup vs baseline: 29.1093x; 1.0003x over previous
import functools

import jax
import jax.numpy as jnp
from jax import lax
from jax.experimental import pallas as pl
from jax.experimental.pallas import tpu as pltpu
from jax.experimental.pallas import tpu_sc as plsc

NC = 2
NS = 16
NW = NC * NS
C = 128

N = 10000
D = 128
DH = D // NC
NPAD = 10240
RPT = NPAD // NS
TPT = N // NS
RZB = 128


def _mesh():
    return plsc.VectorSubcoreMesh(core_axis_name="c", subcore_axis_name="s",
                                  num_cores=NC, num_subcores=NS)


DEG_SLAB = 80


def _sc_deg_body(ei2, zvec_hbm, ones_hbm, out, shared, dst_v, ones_v, zvec_v):
    c = lax.axis_index("c")
    s = lax.axis_index("s")
    wid = s * NC + c
    rows = ei2.shape[1]
    ntail = rows - (NW - 1) * DEG_SLAB
    pltpu.sync_copy(ones_hbm, ones_v)
    pltpu.sync_copy(zvec_hbm, zvec_v)
    pltpu.sync_copy(zvec_v, shared.at[pl.ds(s * RPT, RPT)])

    @pl.when(wid < NW - 1)
    def _():
        pltpu.sync_copy(ei2.at[1, pl.ds(wid * DEG_SLAB, DEG_SLAB)], dst_v)

    @pl.when(wid == NW - 1)
    def _():
        pltpu.sync_copy(ei2.at[1, pl.ds((NW - 1) * DEG_SLAB, ntail)],
                        dst_v.at[pl.ds(0, ntail)])
    plsc.subcore_barrier()

    nj = jnp.where(wid == NW - 1, ntail, DEG_SLAB)

    def chunk(j, carry):
        pltpu.sync_copy(ones_v, shared.at[dst_v.at[j]], add=True)
        return carry

    lax.fori_loop(0, nj, chunk, 0)
    plsc.subcore_barrier()
    pltpu.sync_copy(shared.at[pl.ds(s * RPT, RPT)], zvec_v)
    pltpu.sync_copy(zvec_v, out.at[pl.ds(c * NPAD + s * RPT, RPT)])


def _sc_deg(ei2):
    f = functools.partial(
        pl.kernel,
        out_type=jax.ShapeDtypeStruct((NC * NPAD,), jnp.float32),
        mesh=_mesh(),
        compiler_params=pltpu.CompilerParams(use_tc_tiling_on_sc=False),
        scratch_types=[
            pltpu.VMEM_SHARED((NPAD,), jnp.float32),
            pltpu.VMEM((DEG_SLAB, C), jnp.int32),
            pltpu.VMEM((C,), jnp.float32),
            pltpu.VMEM((RPT,), jnp.float32),
        ],
    )(_sc_deg_body)
    return f(ei2, jnp.zeros((RPT,), jnp.float32), jnp.ones((C,), jnp.float32))


def _sc_edge_body(hp2, idxP, out, tab, acc,
                  r0a, r1a, r0b, r1b, isrcA, idstA, isrcB, idstB,
                  gsemA, gsemB, ssemA, ssemB, isemSA, isemSB, isemDA, isemDB):
    c = lax.axis_index("c")
    s = lax.axis_index("s")
    K = idxP.shape[2]
    P = K // 2

    pltpu.sync_copy(hp2.at[pl.ds(s * RPT, RPT), pl.ds(c * DH, DH)],
                    tab.at[pl.ds(s * RPT, RPT)])
    pltpu.sync_copy(hp2.at[pl.ds(s * RPT, RPT), pl.ds(c * DH, DH)],
                    acc.at[pl.ds(s * RPT, RPT)])
    plsc.subcore_barrier()

    def fire_src(p, isrc, isem):
        pltpu.async_copy(idxP.at[0, s, pl.ds(2 * p, 2)], isrc, isem)

    def fire_dst(p, idst, isem):
        pltpu.async_copy(idxP.at[1, s, pl.ds(2 * p, 2)], idst, isem)

    def w(sem, srcref, dstref):
        pltpu.make_async_copy(srcref, dstref, sem).wait()

    def pair_body(p, r0x, r1x, r0y, r1y, isrcX, idstX, isrcY, idstY,
                  gsemX, gsemY, ssemX, ssemY, isemSX, isemSY,
                  isemDX, isemDY, first):
        j = 2 * p
        w(gsemX, tab.at[isrcX.at[0]], r0x)
        w(gsemX, tab.at[isrcX.at[1]], r1x)
        @pl.when(j + 4 < K)
        def _():
            fire_src(p + 2, isrcX, isemSX)
        w(isemDX, idxP.at[1, s, pl.ds(j, 2)], idstX)
        pltpu.async_copy(r0x, acc.at[idstX.at[0]], ssemX, add=True)
        pltpu.async_copy(r1x, acc.at[idstX.at[1]], ssemX, add=True)
        if not first:
            w(ssemY, r0y, acc.at[idstY.at[0]])
            w(ssemY, r1y, acc.at[idstY.at[1]])
        @pl.when(j + 2 < K)
        def _():
            w(isemSY, idxP.at[0, s, pl.ds(j + 2, 2)], isrcY)
            pltpu.async_copy(tab.at[isrcY.at[0]], r0y, gsemY)
            pltpu.async_copy(tab.at[isrcY.at[1]], r1y, gsemY)
        if not first:
            @pl.when(j + 2 < K)
            def _():
                fire_dst(p + 1, idstY, isemDY)

    A = (r0a, r1a, r0b, r1b, isrcA, idstA, isrcB, idstB,
         gsemA, gsemB, ssemA, ssemB, isemSA, isemSB, isemDA, isemDB)
    B = (r0b, r1b, r0a, r1a, isrcB, idstB, isrcA, idstA,
         gsemB, gsemA, ssemB, ssemA, isemSB, isemSA, isemDB, isemDA)

    fire_src(0, isrcA, isemSA)
    fire_dst(0, idstA, isemDA)
    fire_src(1, isrcB, isemSB)
    fire_dst(1, idstB, isemDB)
    w(isemSA, idxP.at[0, s, pl.ds(0, 2)], isrcA)
    pltpu.async_copy(tab.at[isrcA.at[0]], r0a, gsemA)
    pltpu.async_copy(tab.at[isrcA.at[1]], r1a, gsemA)

    pair_body(0, *A, True)

    def loop(i, carry):
        pair_body(2 * i + 1, *B, False)
        pair_body(2 * i + 2, *A, False)
        return carry

    lax.fori_loop(0, (P - 1) // 2, loop, 0)
    w(ssemA, r0a, acc.at[idstA.at[0]])
    w(ssemA, r1a, acc.at[idstA.at[1]])

    plsc.subcore_barrier()
    rs = (r0a, r1a, r0b, r1b)
    nz = RPT // RZB
    pltpu.async_copy(acc.at[pl.ds(s * RPT, RZB)], rs[0], gsemA)
    for z in range(nz):
        r = s * RPT + z * RZB
        w(gsemA, acc.at[pl.ds(r, RZB)], rs[z % 4])
        if z + 1 < nz:
            if z + 1 >= 4:
                w(ssemA, rs[(z + 1) % 4],
                  out.at[pl.ds(s * RPT, RZB), pl.ds(c * DH, DH)])
            pltpu.async_copy(acc.at[pl.ds(r + RZB, RZB)], rs[(z + 1) % 4],
                             gsemA)
        pltpu.async_copy(rs[z % 4], out.at[pl.ds(r, RZB), pl.ds(c * DH, DH)],
                         ssemA)
    for z in range(min(nz, 4) if nz < 4 else 4):
        w(ssemA, rs[0], out.at[pl.ds(s * RPT, RZB), pl.ds(c * DH, DH)])


def _sc_edge(hp2, idxP):
    f = functools.partial(
        pl.kernel,
        out_type=jax.ShapeDtypeStruct((NPAD, D), jnp.float32),
        mesh=_mesh(),
        compiler_params=pltpu.CompilerParams(use_tc_tiling_on_sc=False),
        scratch_types=[
            pltpu.VMEM_SHARED((NPAD, DH), jnp.float32),
            pltpu.VMEM_SHARED((NPAD, DH), jnp.float32),
            pltpu.VMEM((C, DH), jnp.float32),
            pltpu.VMEM((C, DH), jnp.float32),
            pltpu.VMEM((C, DH), jnp.float32),
            pltpu.VMEM((C, DH), jnp.float32),
            pltpu.VMEM((2, C), jnp.int32),
            pltpu.VMEM((2, C), jnp.int32),
            pltpu.VMEM((2, C), jnp.int32),
            pltpu.VMEM((2, C), jnp.int32),
        ] + [pltpu.SemaphoreType.DMA] * 8,
    )(_sc_edge_body)
    return f(hp2, idxP)


def _dinv_from(degp):
    deg = 1.0 + degp[:N] + degp[NPAD:NPAD + N]
    return lax.rsqrt(deg)[:, None]


def _tc_prep_body(x_ref, w_ref, degp_ref, out_ref):
    h = jnp.dot(x_ref[...], w_ref[...], preferred_element_type=jnp.float32)
    hp = h * _dinv_from(degp_ref[...])
    out_ref[:N, :] = hp
    out_ref[N:, :] = jnp.zeros((NPAD - N, D), jnp.float32)


def _tc_prep(x, W1, degp):
    return pl.pallas_call(
        _tc_prep_body,
        out_shape=jax.ShapeDtypeStruct((NPAD, D), jnp.float32),
        compiler_params=pltpu.CompilerParams(vmem_limit_bytes=100 * 2**20),
    )(x, W1, degp)


def _bn_relu(z, gamma, beta):
    mean = jnp.mean(z, axis=0)
    var = jnp.mean((z - mean[None, :]) ** 2, axis=0)
    y = (z - mean[None, :]) * lax.rsqrt(var + 1e-5)[None, :]
    return jax.nn.relu(y * gamma[None, :] + beta[None, :])


def _tc_mid_body(accp_ref, degp_ref, g_ref, b_ref, w_ref, out_ref):
    dinv = _dinv_from(degp_ref[...])
    z = accp_ref[:N, :] * dinv
    y = _bn_relu(z, g_ref[...], b_ref[...])
    h2 = jnp.dot(y, w_ref[...], preferred_element_type=jnp.float32)
    out_ref[:N, :] = h2 * dinv
    out_ref[N:, :] = jnp.zeros((NPAD - N, D), jnp.float32)


def _tc_mid(accp, degp, gamma, beta, W2):
    return pl.pallas_call(
        _tc_mid_body,
        out_shape=jax.ShapeDtypeStruct((NPAD, D), jnp.float32),
        compiler_params=pltpu.CompilerParams(vmem_limit_bytes=100 * 2**20),
    )(accp, degp, gamma, beta, W2)


def _tc_final_body(accp_ref, degp_ref, g_ref, b_ref, x_ref, out_ref):
    dinv = _dinv_from(degp_ref[...])
    z = accp_ref[:N, :] * dinv
    y = _bn_relu(z, g_ref[...], b_ref[...])
    out_ref[...] = y + x_ref[...]


def _tc_final(accp, degp, gamma, beta, x):
    return pl.pallas_call(
        _tc_final_body,
        out_shape=jax.ShapeDtypeStruct((N, D), jnp.float32),
        compiler_params=pltpu.CompilerParams(vmem_limit_bytes=100 * 2**20),
    )(accp, degp, gamma, beta, x)


def kernel(x, edge_index, W1, b1, gamma1, beta1, W2, b2, gamma2, beta2):
    E = edge_index.shape[1]

    K = -(-E // (NS * C))
    K += (2 - K) % 4
    pad = NS * K * C - E
    padcol = jnp.stack([jnp.zeros((pad,), jnp.int32),
                        jnp.full((pad,), N, jnp.int32)])
    idxP = jnp.concatenate([edge_index, padcol], axis=1).reshape(2, NS, K, C)

    degp = _sc_deg(edge_index.reshape(2, E // C, C))
    hp1 = _tc_prep(x, W1, degp)
    accp1 = _sc_edge(hp1, idxP)
    hp2 = _tc_mid(accp1, degp, gamma1, beta1, W2)
    accp2 = _sc_edge(hp2, idxP)
    return _tc_final(accp2, degp, gamma2, beta2, x)


# speedup vs baseline: 29.1340x; 1.0008x over previous
import functools

import jax
import jax.numpy as jnp
from jax import lax
from jax.experimental import pallas as pl
from jax.experimental.pallas import tpu as pltpu
from jax.experimental.pallas import tpu_sc as plsc

NC = 2
NS = 16
NW = NC * NS
C = 128

N = 10000
D = 128
DH = D // NC
NPAD = 10240
RPT = NPAD // NS
RZB = 128


def _mesh():
    return plsc.VectorSubcoreMesh(core_axis_name="c", subcore_axis_name="s",
                                  num_cores=NC, num_subcores=NS)


DEG_SLAB = 80


def _sc_deg_body(ei2, zvec_hbm, ones_hbm, out, shared, dst_v, ones_v, zvec_v):
    c = lax.axis_index("c")
    s = lax.axis_index("s")
    wid = s * NC + c
    rows = ei2.shape[1]
    ntail = rows - (NW - 1) * DEG_SLAB
    pltpu.sync_copy(ones_hbm, ones_v)
    pltpu.sync_copy(zvec_hbm, zvec_v)
    pltpu.sync_copy(zvec_v, shared.at[pl.ds(s * RPT, RPT)])

    @pl.when(wid < NW - 1)
    def _():
        pltpu.sync_copy(ei2.at[1, pl.ds(wid * DEG_SLAB, DEG_SLAB)], dst_v)

    @pl.when(wid == NW - 1)
    def _():
        pltpu.sync_copy(ei2.at[1, pl.ds((NW - 1) * DEG_SLAB, ntail)],
                        dst_v.at[pl.ds(0, ntail)])
    plsc.subcore_barrier()

    nj = jnp.where(wid == NW - 1, ntail, DEG_SLAB)

    def chunk(j, carry):
        pltpu.sync_copy(ones_v, shared.at[dst_v.at[j]], add=True)
        return carry

    lax.fori_loop(0, nj, chunk, 0)
    plsc.subcore_barrier()
    pltpu.sync_copy(shared.at[pl.ds(s * RPT, RPT)], zvec_v)
    pltpu.sync_copy(zvec_v, out.at[pl.ds(c * NPAD + s * RPT, RPT)])


def _sc_deg(ei2):
    f = functools.partial(
        pl.kernel,
        out_type=jax.ShapeDtypeStruct((NC * NPAD,), jnp.float32),
        mesh=_mesh(),
        compiler_params=pltpu.CompilerParams(use_tc_tiling_on_sc=False),
        scratch_types=[
            pltpu.VMEM_SHARED((NPAD,), jnp.float32),
            pltpu.VMEM((DEG_SLAB, C), jnp.int32),
            pltpu.VMEM((C,), jnp.float32),
            pltpu.VMEM((RPT,), jnp.float32),
        ],
    )(_sc_deg_body)
    return f(ei2, jnp.zeros((RPT,), jnp.float32), jnp.ones((C,), jnp.float32))


def _sc_edge_body(hp2, idxP, out, tab, acc,
                  r0a, r1a, r0b, r1b, isrcA, idstA, isrcB, idstB,
                  gsemA, gsemB, ssemA, ssemB, isemSA, isemSB, isemDA, isemDB):
    c = lax.axis_index("c")
    s = lax.axis_index("s")
    K = idxP.shape[2]
    P = K // 2

    pltpu.sync_copy(hp2.at[pl.ds(s * RPT, RPT), pl.ds(c * DH, DH)],
                    tab.at[pl.ds(s * RPT, RPT)])
    pltpu.sync_copy(hp2.at[pl.ds(s * RPT, RPT), pl.ds(c * DH, DH)],
                    acc.at[pl.ds(s * RPT, RPT)])
    plsc.subcore_barrier()

    def fire_src(p, isrc, isem):
        pltpu.async_copy(idxP.at[0, s, pl.ds(2 * p, 2)], isrc, isem)

    def fire_dst(p, idst, isem):
        pltpu.async_copy(idxP.at[1, s, pl.ds(2 * p, 2)], idst, isem)

    def w(sem, srcref, dstref):
        pltpu.make_async_copy(srcref, dstref, sem).wait()

    def pair_body(p, r0x, r1x, r0y, r1y, isrcX, idstX, isrcY, idstY,
                  gsemX, gsemY, ssemX, ssemY, isemSX, isemSY,
                  isemDX, isemDY, first):
        j = 2 * p
        w(gsemX, tab.at[isrcX.at[0]], r0x)
        w(gsemX, tab.at[isrcX.at[1]], r1x)
        @pl.when(j + 4 < K)
        def _():
            fire_src(p + 2, isrcX, isemSX)
        w(isemDX, idxP.at[1, s, pl.ds(j, 2)], idstX)
        pltpu.async_copy(r0x, acc.at[idstX.at[0]], ssemX, add=True)
        pltpu.async_copy(r1x, acc.at[idstX.at[1]], ssemX, add=True)
        if not first:
            w(ssemY, r0y, acc.at[idstY.at[0]])
            w(ssemY, r1y, acc.at[idstY.at[1]])
        @pl.when(j + 2 < K)
        def _():
            w(isemSY, idxP.at[0, s, pl.ds(j + 2, 2)], isrcY)
            pltpu.async_copy(tab.at[isrcY.at[0]], r0y, gsemY)
            pltpu.async_copy(tab.at[isrcY.at[1]], r1y, gsemY)
        if not first:
            @pl.when(j + 2 < K)
            def _():
                fire_dst(p + 1, idstY, isemDY)

    A = (r0a, r1a, r0b, r1b, isrcA, idstA, isrcB, idstB,
         gsemA, gsemB, ssemA, ssemB, isemSA, isemSB, isemDA, isemDB)
    B = (r0b, r1b, r0a, r1a, isrcB, idstB, isrcA, idstA,
         gsemB, gsemA, ssemB, ssemA, isemSB, isemSA, isemDB, isemDA)

    fire_src(0, isrcA, isemSA)
    fire_dst(0, idstA, isemDA)
    fire_src(1, isrcB, isemSB)
    fire_dst(1, idstB, isemDB)
    w(isemSA, idxP.at[0, s, pl.ds(0, 2)], isrcA)
    pltpu.async_copy(tab.at[isrcA.at[0]], r0a, gsemA)
    pltpu.async_copy(tab.at[isrcA.at[1]], r1a, gsemA)

    pair_body(0, *A, True)

    def loop(i, carry):
        pair_body(2 * i + 1, *B, False)
        pair_body(2 * i + 2, *A, False)
        return carry

    lax.fori_loop(0, (P - 1) // 2, loop, 0)
    w(ssemA, r0a, acc.at[idstA.at[0]])
    w(ssemA, r1a, acc.at[idstA.at[1]])

    plsc.subcore_barrier()
    rs = (r0a, r1a, r0b, r1b)
    nz = RPT // RZB
    pltpu.async_copy(acc.at[pl.ds(s * RPT, RZB)], rs[0], gsemA)
    for z in range(nz):
        r = s * RPT + z * RZB
        w(gsemA, acc.at[pl.ds(r, RZB)], rs[z % 4])
        if z + 1 < nz:
            if z + 1 >= 4:
                w(ssemA, rs[(z + 1) % 4],
                  out.at[pl.ds(s * RPT, RZB), pl.ds(c * DH, DH)])
            pltpu.async_copy(acc.at[pl.ds(r + RZB, RZB)], rs[(z + 1) % 4],
                             gsemA)
        pltpu.async_copy(rs[z % 4], out.at[pl.ds(r, RZB), pl.ds(c * DH, DH)],
                         ssemA)
    for z in range(min(nz, 4) if nz < 4 else 4):
        w(ssemA, rs[0], out.at[pl.ds(s * RPT, RZB), pl.ds(c * DH, DH)])


def _sc_edge(hp2, idxP):
    f = functools.partial(
        pl.kernel,
        out_type=jax.ShapeDtypeStruct((NPAD, D), jnp.float32),
        mesh=_mesh(),
        compiler_params=pltpu.CompilerParams(use_tc_tiling_on_sc=False),
        scratch_types=[
            pltpu.VMEM_SHARED((NPAD, DH), jnp.float32),
            pltpu.VMEM_SHARED((NPAD, DH), jnp.float32),
            pltpu.VMEM((C, DH), jnp.float32),
            pltpu.VMEM((C, DH), jnp.float32),
            pltpu.VMEM((C, DH), jnp.float32),
            pltpu.VMEM((C, DH), jnp.float32),
            pltpu.VMEM((2, C), jnp.int32),
            pltpu.VMEM((2, C), jnp.int32),
            pltpu.VMEM((2, C), jnp.int32),
            pltpu.VMEM((2, C), jnp.int32),
        ] + [pltpu.SemaphoreType.DMA] * 8,
    )(_sc_edge_body)
    return f(hp2, idxP)


def _dinv_from(degp):
    deg = 1.0 + degp[:N] + degp[NPAD:NPAD + N]
    return lax.rsqrt(deg)[:, None]


def _tc_prep_body(x_ref, w_ref, degp_ref, out_ref):
    h = jnp.dot(x_ref[...], w_ref[...], preferred_element_type=jnp.float32)
    hp = h * _dinv_from(degp_ref[...])
    out_ref[:N, :] = hp
    out_ref[N:, :] = jnp.zeros((NPAD - N, D), jnp.float32)


def _tc_prep(x, W1, degp):
    return pl.pallas_call(
        _tc_prep_body,
        out_shape=jax.ShapeDtypeStruct((NPAD, D), jnp.float32),
        compiler_params=pltpu.CompilerParams(vmem_limit_bytes=100 * 2**20),
    )(x, W1, degp)


def _bn_relu(z, gamma, beta):
    mean = jnp.mean(z, axis=0)
    var = jnp.mean((z - mean[None, :]) ** 2, axis=0)
    y = (z - mean[None, :]) * lax.rsqrt(var + 1e-5)[None, :]
    return jax.nn.relu(y * gamma[None, :] + beta[None, :])


def _tc_mid_body(accp_ref, degp_ref, g_ref, b_ref, w_ref, out_ref):
    dinv = _dinv_from(degp_ref[...])
    z = accp_ref[:N, :] * dinv
    y = _bn_relu(z, g_ref[...], b_ref[...])
    h2 = jnp.dot(y, w_ref[...], preferred_element_type=jnp.float32)
    out_ref[:N, :] = h2 * dinv
    out_ref[N:, :] = jnp.zeros((NPAD - N, D), jnp.float32)


def _tc_mid(accp, degp, gamma, beta, W2):
    return pl.pallas_call(
        _tc_mid_body,
        out_shape=jax.ShapeDtypeStruct((NPAD, D), jnp.float32),
        compiler_params=pltpu.CompilerParams(vmem_limit_bytes=100 * 2**20),
    )(accp, degp, gamma, beta, W2)


def _tc_final_body(accp_ref, degp_ref, g_ref, b_ref, x_ref, out_ref):
    dinv = _dinv_from(degp_ref[...])
    z = accp_ref[:N, :] * dinv
    y = _bn_relu(z, g_ref[...], b_ref[...])
    out_ref[...] = y + x_ref[...]


def _tc_final(accp, degp, gamma, beta, x):
    return pl.pallas_call(
        _tc_final_body,
        out_shape=jax.ShapeDtypeStruct((N, D), jnp.float32),
        compiler_params=pltpu.CompilerParams(vmem_limit_bytes=100 * 2**20),
    )(accp, degp, gamma, beta, x)


def kernel(x, edge_index, W1, b1, gamma1, beta1, W2, b2, gamma2, beta2):
    E = edge_index.shape[1]

    K = -(-E // (NS * C))
    K += (2 - K) % 4
    pad = NS * K * C - E
    padcol = jnp.stack([jnp.zeros((pad,), jnp.int32),
                        jnp.full((pad,), N, jnp.int32)])
    idxP = jnp.concatenate([edge_index, padcol], axis=1).reshape(2, NS, K, C)

    degp = _sc_deg(edge_index.reshape(2, E // C, C))
    hp1 = _tc_prep(x, W1, degp)
    accp1 = _sc_edge(hp1, idxP)
    hp2 = _tc_mid(accp1, degp, gamma1, beta1, W2)
    accp2 = _sc_edge(hp2, idxP)
    return _tc_final(accp2, degp, gamma2, beta2, x)
